# baseline (speedup 1.0000x reference)
import functools

import jax
import jax.numpy as jnp
from jax import lax
from jax.experimental import pallas as pl
from jax.experimental.pallas import tpu as pltpu
from jax.experimental.pallas import tpu_sc as plsc

_NUM_CLASSES = 1000000
_HIDDEN = 32
_BATCH = 16384

_NC = 2
_NS = 16
_L = 16
_NW = _NC * _NS
_B_PER_W = _BATCH // _NW
_NGRP = _B_PER_W // _L
_SUB = 8
_BANDS = _HIDDEN // _SUB
_STEPS = _NGRP * _BANDS
_NBUF = 6
_AHEAD = 4
_DROPCOL = _NUM_CLASSES >> 7
_DROPLANE = _NUM_CLASSES & 127

_mesh = plsc.VectorSubcoreMesh(core_axis_name="c", subcore_axis_name="s")


@functools.partial(
    pl.kernel,
    mesh=_mesh,
    compiler_params=pltpu.CompilerParams(needs_layout_passes=False),
    out_type=jax.ShapeDtypeStruct((_BANDS, _SUB, _BATCH), jnp.float32),
    scratch_types=[
        pltpu.VMEM((_B_PER_W,), jnp.int32),
        pltpu.VMEM((_B_PER_W,), jnp.int32),
        pltpu.VMEM((_B_PER_W,), jnp.int32),
        pltpu.VMEM((_BANDS, _SUB, 128), jnp.float32),
        pltpu.VMEM((_BANDS, _SUB, _L), jnp.float32),
        pltpu.VMEM((_NBUF, _L, _SUB, 128), jnp.float32),
        pltpu.VMEM((_BANDS, _SUB, _B_PER_W), jnp.float32),
        pltpu.SemaphoreType.DMA((_NBUF,)),
    ],
)
def _embed(labels_hbm, drop_hbm, table_hbm, out_hbm,
           lab_v, drop_v, idx_v, dstage, dval, stage_v, rows_v, sem):
    wid = lax.axis_index("s") * _NC + lax.axis_index("c")
    base = wid * _B_PER_W

    pltpu.sync_copy(labels_hbm.at[pl.ds(base, _B_PER_W)], lab_v)
    pltpu.sync_copy(drop_hbm.at[pl.ds(base, _B_PER_W)], drop_v)

    drop_idx = jnp.full((_L,), _NUM_CLASSES, jnp.int32)
    lane_iota = lax.iota(jnp.int32, _L)

    dcol_start = jnp.full((), _DROPCOL * 128, jnp.int32)
    pltpu.sync_copy(table_hbm.at[:, :, pl.ds(dcol_start, 128)], dstage)
    for a in range(_BANDS):
        for b in range(_SUB):
            dval[a, b, pl.ds(0, _L)] = plsc.load_gather(
                dstage,
                [jnp.full((_L,), a, jnp.int32),
                 jnp.full((_L,), b, jnp.int32),
                 jnp.full((_L,), _DROPLANE, jnp.int32)],
            )

    for k in range(_NGRP):
        sl = pl.ds(k * _L, _L)
        idx_v[sl] = jnp.where(drop_v[sl] != 0, drop_idx, lab_v[sl])

    def fire(s):
        s = jnp.minimum(s, _STEPS - 1)
        g = s >> 2
        p = s & 3
        buf = lax.rem(s, _NBUF)
        sl = pl.ds(g * _L, _L)
        tcol = idx_v[sl] >> 7
        dr = drop_v[sl]
        for l in range(_L):
            @pl.when(dr[l] == 0)
            def _():
                pltpu.async_copy(
                    table_hbm.at[p, :, pl.ds(tcol[l] * 128, 128)],
                    stage_v.at[buf, l],
                    sem.at[buf],
                )

    def fired_count(s):
        g = s >> 2
        keep = (drop_v[pl.ds(g * _L, _L)] == 0).astype(jnp.int32)
        return jnp.sum(keep)

    def drain(n, buf):
        def wbody(i, c):
            pltpu.make_async_copy(
                table_hbm.at[0, :, pl.ds(0, 128)],
                stage_v.at[0, 0],
                sem.at[buf],
            ).wait()
            return c
        lax.fori_loop(0, n, wbody, 0)

    def select(s):
        g = s >> 2
        p = s & 3
        buf = lax.rem(s, _NBUF)
        sl = pl.ds(g * _L, _L)
        lane = idx_v[sl] & 127
        mask = drop_v[sl] != 0
        p_f = jnp.full((_L,), 0, jnp.int32) + p
        stg = stage_v.at[buf]
        for b in range(_SUB):
            val = plsc.load_gather(
                stg,
                [lane_iota,
                 jnp.full((_L,), b, jnp.int32),
                 lane],
            )
            dv = plsc.load_gather(
                dval,
                [p_f, jnp.full((_L,), b, jnp.int32), lane_iota],
            )
            rows_v[p, b, sl] = jnp.where(mask, dv, val)

    for i in range(_AHEAD):
        fire(jnp.int32(i))

    def step(s):
        fire(s + _AHEAD)
        drain(fired_count(s), lax.rem(s, _NBUF))
        select(s)

    pl.loop(0, _STEPS, unroll=2)(step)

    drain(_AHEAD * fired_count(jnp.int32(_STEPS - 1)),
          lax.rem(jnp.int32(_STEPS - 1), _NBUF))

    pltpu.sync_copy(rows_v, out_hbm.at[:, :, pl.ds(base, _B_PER_W)])


def kernel(labels, force_drop_ids, table):
    lab = labels.astype(jnp.int32)
    drop = force_drop_ids.astype(jnp.int32)
    table_t = table.T.reshape(_BANDS, _SUB, _NUM_CLASSES + 1)
    out_t = _embed(lab, drop, table_t)
    return out_t.reshape(_HIDDEN, _BATCH).T


# speedup vs baseline: 1.0024x; 1.0024x over previous
import functools

import jax
import jax.numpy as jnp
from jax import lax
from jax.experimental import pallas as pl
from jax.experimental.pallas import tpu as pltpu
from jax.experimental.pallas import tpu_sc as plsc

_NUM_CLASSES = 1000000
_HIDDEN = 32
_BATCH = 16384

_NC = 2
_NS = 16
_L = 16
_NW = _NC * _NS
_B_PER_W = _BATCH // _NW
_NGRP = _B_PER_W // _L
_SUB = 8
_BANDS = _HIDDEN // _SUB
_STEPS = _NGRP * _BANDS
_NBUF = 6
_AHEAD = 5
_DROPCOL = _NUM_CLASSES >> 7
_DROPLANE = _NUM_CLASSES & 127

_mesh = plsc.VectorSubcoreMesh(core_axis_name="c", subcore_axis_name="s")


@functools.partial(
    pl.kernel,
    mesh=_mesh,
    compiler_params=pltpu.CompilerParams(needs_layout_passes=False),
    out_type=jax.ShapeDtypeStruct((_BANDS, _SUB, _BATCH), jnp.float32),
    scratch_types=[
        pltpu.VMEM((_B_PER_W,), jnp.int32),
        pltpu.VMEM((_B_PER_W,), jnp.int32),
        pltpu.VMEM((_B_PER_W,), jnp.int32),
        pltpu.VMEM((_BANDS, _SUB, 128), jnp.float32),
        pltpu.VMEM((_BANDS, _SUB, _L), jnp.float32),
        pltpu.VMEM((_NBUF, _L, _SUB, 128), jnp.float32),
        pltpu.VMEM((_BANDS, _SUB, _B_PER_W), jnp.float32),
        pltpu.SemaphoreType.DMA((_NBUF,)),
    ],
)
def _embed(labels_hbm, drop_hbm, table_hbm, out_hbm,
           lab_v, drop_v, idx_v, dstage, dval, stage_v, rows_v, sem):
    wid = lax.axis_index("s") * _NC + lax.axis_index("c")
    base = wid * _B_PER_W

    pltpu.sync_copy(labels_hbm.at[pl.ds(base, _B_PER_W)], lab_v)
    pltpu.sync_copy(drop_hbm.at[pl.ds(base, _B_PER_W)], drop_v)

    drop_idx = jnp.full((_L,), _NUM_CLASSES, jnp.int32)
    lane_iota = lax.iota(jnp.int32, _L)

    dcol_start = jnp.full((), _DROPCOL * 128, jnp.int32)
    pltpu.sync_copy(table_hbm.at[:, :, pl.ds(dcol_start, 128)], dstage)
    for a in range(_BANDS):
        for b in range(_SUB):
            dval[a, b, pl.ds(0, _L)] = plsc.load_gather(
                dstage,
                [jnp.full((_L,), a, jnp.int32),
                 jnp.full((_L,), b, jnp.int32),
                 jnp.full((_L,), _DROPLANE, jnp.int32)],
            )

    for k in range(_NGRP):
        sl = pl.ds(k * _L, _L)
        idx_v[sl] = jnp.where(drop_v[sl] != 0, drop_idx, lab_v[sl])

    def fire(s):
        s = jnp.minimum(s, _STEPS - 1)
        g = s >> 2
        p = s & 3
        buf = lax.rem(s, _NBUF)
        sl = pl.ds(g * _L, _L)
        tcol = idx_v[sl] >> 7
        dr = drop_v[sl]
        for l in range(_L):
            @pl.when(dr[l] == 0)
            def _():
                pltpu.async_copy(
                    table_hbm.at[p, :, pl.ds(tcol[l] * 128, 128)],
                    stage_v.at[buf, l],
                    sem.at[buf],
                )

    def fired_count(s):
        g = s >> 2
        keep = (drop_v[pl.ds(g * _L, _L)] == 0).astype(jnp.int32)
        return jnp.sum(keep)

    def drain(n, buf):
        def wbody(i, c):
            pltpu.make_async_copy(
                table_hbm.at[0, :, pl.ds(0, 128)],
                stage_v.at[0, 0],
                sem.at[buf],
            ).wait()
            return c
        lax.fori_loop(0, n, wbody, 0)

    def select(s):
        g = s >> 2
        p = s & 3
        buf = lax.rem(s, _NBUF)
        sl = pl.ds(g * _L, _L)
        lane = idx_v[sl] & 127
        mask = drop_v[sl] != 0
        p_f = jnp.full((_L,), 0, jnp.int32) + p
        stg = stage_v.at[buf]
        for b in range(_SUB):
            val = plsc.load_gather(
                stg,
                [lane_iota,
                 jnp.full((_L,), b, jnp.int32),
                 lane],
            )
            dv = plsc.load_gather(
                dval,
                [p_f, jnp.full((_L,), b, jnp.int32), lane_iota],
            )
            rows_v[p, b, sl] = jnp.where(mask, dv, val)

    for i in range(_AHEAD):
        fire(jnp.int32(i))

    def step(s):
        fire(s + _AHEAD)
        drain(fired_count(s), lax.rem(s, _NBUF))
        select(s)

    pl.loop(0, _STEPS, unroll=2)(step)

    drain(_AHEAD * fired_count(jnp.int32(_STEPS - 1)),
          lax.rem(jnp.int32(_STEPS - 1), _NBUF))

    pltpu.sync_copy(rows_v, out_hbm.at[:, :, pl.ds(base, _B_PER_W)])


def kernel(labels, force_drop_ids, table):
    lab = labels.astype(jnp.int32)
    drop = force_drop_ids.astype(jnp.int32)
    table_t = table.T.reshape(_BANDS, _SUB, _NUM_CLASSES + 1)
    out_t = _embed(lab, drop, table_t)
    return out_t.reshape(_HIDDEN, _BATCH).T
